# baseline (speedup 1.0000x reference)
import jax
import jax.numpy as jnp
from jax import lax
from jax.experimental import pallas as pl
from jax.experimental.pallas import tpu as pltpu
from jax.experimental.pallas import tpu_sc as plsc

BATCH = 4096
GROUPS = 512
OUT_DIM = 8
IN_W = 2 * GROUPS
OUT_W = OUT_DIM * GROUPS
NC, NS, L = 2, 16, 16
NW = NC * NS
ROWS_PER_W = BATCH // NW
NCHUNK = GROUPS // L
BR = 4


def _compute_block(xblk, oblk, pref):
    @plsc.parallel_loop(0, BR * NCHUNK, unroll=2)
    def chunk(cc):
        r4 = cc >> 5
        cl = cc & (NCHUNK - 1)
        iota = lax.iota(jnp.int32, L)
        gvec = cl * L + iota
        a0 = plsc.load_gather(xblk, [r4 * IN_W + 2 * gvec])
        a1 = plsc.load_gather(xblk, [r4 * IN_W + 2 * gvec + 1])
        abs0 = jnp.abs(a0)
        abs1 = jnp.abs(a1)
        swap = abs1 < abs0
        s0 = jnp.minimum(abs0, abs1)
        s1 = jnp.maximum(abs0, abs1)
        c0 = s0
        c1 = s1 - s0
        asmall = jnp.where(swap, a1, a0)
        alarge = jnp.where(swap, a0, a1)
        sg0 = (asmall >= 0).astype(jnp.int32)
        sg1 = (alarge >= 0).astype(jnp.int32)
        swi = swap.astype(jnp.int32)
        m1 = jnp.where(alarge >= 0, 3 - swi, swi)
        m0 = sg0 + sg1 + jnp.where(swap, sg0, sg1)
        b0 = gvec * 64 + m0 * 8
        b1 = gvec * 64 + 32 + m1 * 8
        ov = r4 * OUT_W + 8 * gvec
        for d in range(OUT_DIM):
            p0 = plsc.load_gather(pref, [b0 + d])
            p1 = plsc.load_gather(pref, [b1 + d])
            o = c0 * p0 + c1 * p1
            plsc.store_scatter(oblk, [ov + d], o)


def _sc_kernel(x_hbm, p_hbm, out_hbm, xb0, xb1, ob0, ob1, pbuf,
               psem, isem0, isem1, osem0, osem1):
    wid = lax.axis_index("s") * NC + lax.axis_index("c")
    rbase = wid * ROWS_PER_W
    xbs, obs = (xb0, xb1), (ob0, ob1)
    isems, osems = (isem0, isem1), (osem0, osem1)

    def in_copy(r, p):
        return pltpu.make_async_copy(
            x_hbm.at[pl.ds(r * IN_W, BR * IN_W)], xbs[p], isems[p])

    def out_copy(r, p):
        return pltpu.make_async_copy(
            obs[p], out_hbm.at[pl.ds(r * OUT_W, BR * OUT_W)], osems[p])

    pcopy = pltpu.make_async_copy(p_hbm, pbuf, psem)
    pcopy.start()
    in_copy(rbase, 0).start()
    in_copy(rbase + BR, 1).start()
    pcopy.wait()

    nit = ROWS_PER_W // (2 * BR)

    def block_pair(rr, carry):
        for p in range(2):
            r = rbase + (2 * rr + p) * BR
            in_copy(r, p).wait()

            @pl.when(rr > 0)
            def _():
                out_copy(r - 2 * BR, p).wait()

            _compute_block(xbs[p], obs[p], pbuf)

            @pl.when(rr < nit - 1)
            def _():
                in_copy(r + 2 * BR, p).start()

            out_copy(r, p).start()
        return carry

    lax.fori_loop(0, nit, block_pair, jnp.int32(0))
    out_copy(rbase + ROWS_PER_W - 2 * BR, 0).wait()
    out_copy(rbase + ROWS_PER_W - BR, 1).wait()


@jax.jit
def _run(x_flat, p_packed):
    mesh = plsc.VectorSubcoreMesh(core_axis_name="c", subcore_axis_name="s")
    f = pl.kernel(
        _sc_kernel,
        out_type=jax.ShapeDtypeStruct((BATCH * OUT_W,), jnp.float32),
        mesh=mesh,
        scratch_types=[
            pltpu.VMEM((BR * IN_W,), jnp.float32),
            pltpu.VMEM((BR * IN_W,), jnp.float32),
            pltpu.VMEM((BR * OUT_W,), jnp.float32),
            pltpu.VMEM((BR * OUT_W,), jnp.float32),
            pltpu.VMEM((GROUPS * 64,), jnp.float32),
            pltpu.SemaphoreType.DMA,
            pltpu.SemaphoreType.DMA,
            pltpu.SemaphoreType.DMA,
            pltpu.SemaphoreType.DMA,
            pltpu.SemaphoreType.DMA,
        ],
        compiler_params=pltpu.CompilerParams(
            needs_layout_passes=False, use_tc_tiling_on_sc=True),
    )
    return f(x_flat, p_packed)


def kernel(X, params):
    p_packed = params[:, jnp.array([0, 2, 6, 8, 1, 3, 5, 7]), :].reshape(-1)
    out_flat = _run(X.reshape(-1), p_packed)
    return out_flat.reshape(BATCH, OUT_W)


# speedup vs baseline: 1.2625x; 1.2625x over previous
import jax
import jax.numpy as jnp
from jax import lax
from jax.experimental import pallas as pl
from jax.experimental.pallas import tpu as pltpu
from jax.experimental.pallas import tpu_sc as plsc

BATCH = 4096
GROUPS = 512
OUT_DIM = 8
IN_W = 2 * GROUPS
OUT_W = OUT_DIM * GROUPS
NC, NS, L = 2, 16, 16
NW = NC * NS
ROWS_PER_W = BATCH // NW
NCHUNK = GROUPS // L
BR = 8


def _compute_block(xblk, oblk, pref):
    @plsc.parallel_loop(0, BR * NCHUNK, unroll=1)
    def chunk(cc):
        r4 = cc >> 5
        cl = cc & (NCHUNK - 1)
        iota = lax.iota(jnp.int32, L)
        gvec = cl * L + iota
        a0 = plsc.load_gather(xblk, [r4 * IN_W + 2 * gvec])
        a1 = plsc.load_gather(xblk, [r4 * IN_W + 2 * gvec + 1])
        abs0 = jnp.abs(a0)
        abs1 = jnp.abs(a1)
        swap = abs1 < abs0
        s0 = jnp.minimum(abs0, abs1)
        s1 = jnp.maximum(abs0, abs1)
        c0 = s0
        c1 = s1 - s0
        asmall = jnp.where(swap, a1, a0)
        alarge = jnp.where(swap, a0, a1)
        sg0 = (asmall >= 0).astype(jnp.int32)
        sg1 = (alarge >= 0).astype(jnp.int32)
        swi = swap.astype(jnp.int32)
        m1 = jnp.where(alarge >= 0, 3 - swi, swi)
        m0 = sg0 + sg1 + jnp.where(swap, sg0, sg1)
        b0 = gvec * 64 + m0 * 8
        b1 = gvec * 64 + 32 + m1 * 8
        ov = r4 * OUT_W + 8 * gvec
        for d in range(OUT_DIM):
            p0 = plsc.load_gather(pref, [b0 + d])
            p1 = plsc.load_gather(pref, [b1 + d])
            o = c0 * p0 + c1 * p1
            plsc.store_scatter(oblk, [ov + d], o)


def _sc_kernel(x_hbm, p_hbm, out_hbm, xb0, xb1, ob0, ob1, pbuf,
               psem, isem0, isem1, osem0, osem1):
    wid = lax.axis_index("s") * NC + lax.axis_index("c")
    rbase = wid * ROWS_PER_W
    xbs, obs = (xb0, xb1), (ob0, ob1)
    isems, osems = (isem0, isem1), (osem0, osem1)

    def in_copy(r, p):
        return pltpu.make_async_copy(
            x_hbm.at[pl.ds(r * IN_W, BR * IN_W)], xbs[p], isems[p])

    def out_copy(r, p):
        return pltpu.make_async_copy(
            obs[p], out_hbm.at[pl.ds(r * OUT_W, BR * OUT_W)], osems[p])

    pcopy = pltpu.make_async_copy(p_hbm, pbuf, psem)
    pcopy.start()
    in_copy(rbase, 0).start()
    in_copy(rbase + BR, 1).start()
    pcopy.wait()

    nit = ROWS_PER_W // (2 * BR)

    def block_pair(rr, carry):
        for p in range(2):
            r = rbase + (2 * rr + p) * BR
            in_copy(r, p).wait()

            @pl.when(rr > 0)
            def _():
                out_copy(r - 2 * BR, p).wait()

            _compute_block(xbs[p], obs[p], pbuf)

            @pl.when(rr < nit - 1)
            def _():
                in_copy(r + 2 * BR, p).start()

            out_copy(r, p).start()
        return carry

    lax.fori_loop(0, nit, block_pair, jnp.int32(0))
    out_copy(rbase + ROWS_PER_W - 2 * BR, 0).wait()
    out_copy(rbase + ROWS_PER_W - BR, 1).wait()


@jax.jit
def _run(x_flat, p_packed):
    mesh = plsc.VectorSubcoreMesh(core_axis_name="c", subcore_axis_name="s")
    f = pl.kernel(
        _sc_kernel,
        out_type=jax.ShapeDtypeStruct((BATCH * OUT_W,), jnp.float32),
        mesh=mesh,
        scratch_types=[
            pltpu.VMEM((BR * IN_W,), jnp.float32),
            pltpu.VMEM((BR * IN_W,), jnp.float32),
            pltpu.VMEM((BR * OUT_W,), jnp.float32),
            pltpu.VMEM((BR * OUT_W,), jnp.float32),
            pltpu.VMEM((GROUPS * 64,), jnp.float32),
            pltpu.SemaphoreType.DMA,
            pltpu.SemaphoreType.DMA,
            pltpu.SemaphoreType.DMA,
            pltpu.SemaphoreType.DMA,
            pltpu.SemaphoreType.DMA,
        ],
        compiler_params=pltpu.CompilerParams(
            needs_layout_passes=False, use_tc_tiling_on_sc=True),
    )
    return f(x_flat, p_packed)


def kernel(X, params):
    p_packed = params[:, jnp.array([0, 2, 6, 8, 1, 3, 5, 7]), :].reshape(-1)
    out_flat = _run(X.reshape(-1), p_packed)
    return out_flat.reshape(BATCH, OUT_W)
